# baseline (speedup 1.0000x reference)
import jax
import jax.numpy as jnp
from jax.experimental import pallas as pl

N = 16384
D_IN = 64
D_HID = 64
D_OUT = 128
RATIO = 0.5
N_SAMPLE = int(N * RATIO)
RADIUS = 0.2
MAX_NEIGHBORS = 32
CHUNK = 1024


def _fps(pos, n_sample):
    n = pos.shape[0]
    sel = jnp.zeros((n_sample,), dtype=jnp.int32)
    dists = jnp.full((n,), jnp.inf, dtype=jnp.float32)

    def body(i, carry):
        sel, dists = carry
        last = sel[i - 1]
        d = jnp.sum((pos - pos[last]) ** 2, axis=-1)
        dists = jnp.minimum(dists, d)
        nxt = jnp.argmax(dists).astype(jnp.int32)
        sel = sel.at[i].set(nxt)
        return (sel, dists)

    sel, _ = jax.lax.fori_loop(1, n_sample, body, (sel, dists))
    return sel


def _radius_neighbors(pos_all, pos_s):
    r2 = RADIUS * RADIUS

    def chunk_fn(ps):
        d2 = jnp.sum((ps[:, None, :] - pos_all[None, :, :]) ** 2, axis=-1)
        score = jnp.where(d2 <= r2, -d2, -jnp.inf)
        vals, cols = jax.lax.top_k(score, MAX_NEIGHBORS)
        return vals, cols

    vals, cols = jax.lax.map(chunk_fn, pos_s.reshape(-1, CHUNK, 3))
    return vals.reshape(-1, MAX_NEIGHBORS), cols.reshape(-1, MAX_NEIGHBORS)


def kernel(features, pos, batch, W1, b1, W2, b2):
    idx = _fps(pos, N_SAMPLE)
    pos_s = pos[idx]
    vals, cols = _radius_neighbors(pos, pos_s)
    valid = jnp.isfinite(vals)
    x_j = features[cols]
    rel = pos[cols] - pos_s[:, None, :]
    msg = jnp.concatenate([x_j, rel], axis=-1)
    h = jax.nn.relu(msg @ W1 + b1)
    h = jax.nn.relu(h @ W2 + b2)
    h = jnp.where(valid[:, :, None], h, -jnp.inf)
    out = jnp.max(h, axis=1)
    out = jnp.where(jnp.isfinite(out), out, 0.0)
    return out, pos_s, batch[idx]


# speedup vs baseline: 3.8553x; 3.8553x over previous
import functools

import jax
import jax.numpy as jnp
from jax.experimental import pallas as pl
from jax.experimental.pallas import tpu as pltpu

N = 16384
D_IN = 64
D_HID = 64
D_OUT = 128
RATIO = 0.5
N_SAMPLE = int(N * RATIO)
RADIUS = 0.2
MAX_NEIGHBORS = 32
CHUNK = 1024

_R = 128
_BIG = 1 << 30


def _fps_body(x_ref, y_ref, z_ref, sel_ref, sx_ref, sy_ref, sz_ref, n_iter):
    x = x_ref[:]
    y = y_ref[:]
    z = z_ref[:]
    iota = (jax.lax.broadcasted_iota(jnp.int32, (_R, _R), 0) * _R
            + jax.lax.broadcasted_iota(jnp.int32, (_R, _R), 1))

    sel_ref[0:1, :] = jnp.zeros((1, 1), jnp.int32)
    sx_ref[0:1, :] = x_ref[0:1, 0:1]
    sy_ref[0:1, :] = y_ref[0:1, 0:1]
    sz_ref[0:1, :] = z_ref[0:1, 0:1]

    def body(i, carry):
        dists, cx, cy, cz = carry
        dx = x - cx
        dy = y - cy
        dz = z - cz
        d = (dx * dx + dz * dz) + dy * dy
        dists = jnp.minimum(dists, d)
        m = jnp.max(dists)
        masked = jnp.where(dists == m, iota, _BIG)
        nxt = jnp.min(masked)
        pick = masked == nxt
        cx = jnp.sum(jnp.where(pick, x, 0.0))
        cy = jnp.sum(jnp.where(pick, y, 0.0))
        cz = jnp.sum(jnp.where(pick, z, 0.0))
        sel_ref[pl.ds(i, 1), :] = jnp.full((1, 1), nxt, jnp.int32)
        sx_ref[pl.ds(i, 1), :] = jnp.full((1, 1), cx, jnp.float32)
        sy_ref[pl.ds(i, 1), :] = jnp.full((1, 1), cy, jnp.float32)
        sz_ref[pl.ds(i, 1), :] = jnp.full((1, 1), cz, jnp.float32)
        return dists, cx, cy, cz

    init = (jnp.full((_R, _R), jnp.inf, jnp.float32),
            x_ref[0, 0], y_ref[0, 0], z_ref[0, 0])
    jax.lax.fori_loop(1, n_iter, body, init)


def _fps_pallas(pos, n_sample):
    xs = pos[:, 0].reshape(_R, _R)
    ys = pos[:, 1].reshape(_R, _R)
    zs = pos[:, 2].reshape(_R, _R)
    sel, sx, sy, sz = pl.pallas_call(
        functools.partial(_fps_body, n_iter=n_sample),
        out_shape=(
            jax.ShapeDtypeStruct((n_sample, 1), jnp.int32),
            jax.ShapeDtypeStruct((n_sample, 1), jnp.float32),
            jax.ShapeDtypeStruct((n_sample, 1), jnp.float32),
            jax.ShapeDtypeStruct((n_sample, 1), jnp.float32),
        ),
    )(xs, ys, zs)
    idx = sel[:, 0]
    pos_s = jnp.concatenate([sx, sy, sz], axis=1)
    return idx, pos_s


def _radius_neighbors(pos_all, pos_s):
    r2 = RADIUS * RADIUS

    def chunk_fn(ps):
        d2 = jnp.sum((ps[:, None, :] - pos_all[None, :, :]) ** 2, axis=-1)
        score = jnp.where(d2 <= r2, -d2, -jnp.inf)
        vals, cols = jax.lax.top_k(score, MAX_NEIGHBORS)
        return vals, cols

    vals, cols = jax.lax.map(chunk_fn, pos_s.reshape(-1, CHUNK, 3))
    return vals.reshape(-1, MAX_NEIGHBORS), cols.reshape(-1, MAX_NEIGHBORS)


def kernel(features, pos, batch, W1, b1, W2, b2):
    idx, pos_s = _fps_pallas(pos, N_SAMPLE)
    vals, cols = _radius_neighbors(pos, pos_s)
    valid = jnp.isfinite(vals)
    x_j = features[cols]
    rel = pos[cols] - pos_s[:, None, :]
    msg = jnp.concatenate([x_j, rel], axis=-1)
    h = jax.nn.relu(msg @ W1 + b1)
    h = jax.nn.relu(h @ W2 + b2)
    h = jnp.where(valid[:, :, None], h, -jnp.inf)
    out = jnp.max(h, axis=1)
    out = jnp.where(jnp.isfinite(out), out, 0.0)
    return out, pos_s, batch[idx]
